# Initial kernel scaffold; baseline (speedup 1.0000x reference)
import jax
import jax.numpy as jnp
from jax.experimental import pallas as pl


def kernel(x, table):
    raise NotImplementedError("write your pallas kernel here")


# speedup vs baseline: 1.8774x; 1.8774x over previous
import functools

import jax
import jax.numpy as jnp
from jax import lax
from jax.experimental import pallas as pl
from jax.experimental.pallas import tpu as pltpu
from jax.experimental.pallas import tpu_sc as plsc

VOCAB = 1000000
EMBED_DIM = 64
BATCH = 16384
HIST_LEN = 50

NC = 2
NS = 16
NW = NC * NS

B_TOTAL = BATCH * HIST_LEN
B_PER_W = B_TOTAL // NW
CHUNK = 128
N_CHUNKS = B_PER_W // CHUNK
NBUF = 4


def _gather_kernel(x_hbm, table_hbm, out_hbm, idx_v, rows_v, *gsems):
    wid = lax.axis_index("s") * NC + lax.axis_index("c")
    base = wid * B_PER_W

    pltpu.sync_copy(x_hbm.at[wid], idx_v)

    for b in range(NBUF):
        pltpu.async_copy(table_hbm.at[idx_v.at[b]], rows_v.at[b], gsems[b])

    @pl.loop(0, N_CHUNKS - NBUF, step=NBUF)
    def _(g):
        for b in range(NBUF):
            j = g + b
            pltpu.make_async_copy(table_hbm.at[idx_v.at[j]], rows_v.at[b],
                                  gsems[b]).wait()
            pltpu.sync_copy(rows_v.at[b],
                            out_hbm.at[pl.ds(base + j * CHUNK, CHUNK)])
            pltpu.async_copy(table_hbm.at[idx_v.at[j + NBUF]], rows_v.at[b],
                             gsems[b])

    for b in range(NBUF):
        j = N_CHUNKS - NBUF + b
        pltpu.make_async_copy(table_hbm.at[idx_v.at[j]], rows_v.at[b],
                              gsems[b]).wait()
        pltpu.sync_copy(rows_v.at[b],
                        out_hbm.at[pl.ds(base + j * CHUNK, CHUNK)])


@jax.jit
def kernel(x, table):
    x_flat = x.reshape(NW, N_CHUNKS, CHUNK).astype(jnp.int32)
    mesh = plsc.VectorSubcoreMesh(core_axis_name="c", subcore_axis_name="s")
    out = pl.kernel(
        _gather_kernel,
        out_type=jax.ShapeDtypeStruct((B_TOTAL, EMBED_DIM), jnp.float32),
        mesh=mesh,
        scratch_types=[
            pltpu.VMEM((N_CHUNKS, CHUNK), jnp.int32),
            pltpu.VMEM((NBUF, CHUNK, EMBED_DIM), jnp.float32),
        ] + [pltpu.SemaphoreType.DMA] * NBUF,
        compiler_params=pltpu.CompilerParams(use_tc_tiling_on_sc=False),
    )(x_flat, table)
    return out.reshape(BATCH, HIST_LEN, EMBED_DIM)
